# baseline (speedup 1.0000x reference)
import functools

import jax
import jax.numpy as jnp
from jax import lax
from jax.experimental import pallas as pl
from jax.experimental.pallas import tpu as pltpu
from jax.experimental.pallas import tpu_sc as plsc

N = 10000
D = 128
H = 128
GATE = 3 * H

NUM_CORES = 2
NUM_TILES = 16
HALF = N // 2
CHUNK = 64
CHUNKS = 640
GROUP = 40
NGROUPS = CHUNKS // GROUP
MSTEPS = (GROUP - 4) // 4
E_PAD = NUM_TILES * CHUNKS * CHUNK // 2
ACC_HALF = 5056
ACC_ROWS = 2 * ACC_HALF
ROWS_PER_TILE = ACC_ROWS // NUM_TILES
DUMP_SPREAD = 56

ROW_BLOCK = 200
BLOCKS_PER_CORE = HALF // ROW_BLOCK
H_BLOCK_OFF = ACC_HALF // ROW_BLOCK


def _sc_agg_build():
  mesh = plsc.VectorSubcoreMesh(core_axis_name="c", subcore_axis_name="s")

  @functools.partial(
      pl.kernel,
      out_type=jax.ShapeDtypeStruct((NUM_CORES, ACC_ROWS, D), jnp.float32),
      mesh=mesh,
      scratch_types=[
          pltpu.VMEM_SHARED((ACC_ROWS, D), jnp.float32),
          pltpu.VMEM((GROUP, CHUNK), jnp.int32),
          pltpu.VMEM((GROUP, CHUNK), jnp.int32),
          [pltpu.VMEM((CHUNK, D), jnp.float32)] * 4,
          [pltpu.SemaphoreType.DMA] * 4,
          [pltpu.SemaphoreType.DMA] * 4,
      ],
  )
  def sc_agg(tab_hbm, src_hbm, dst_hbm, zeros_hbm, out_hbm,
             acc, src_v, dst_v, rows, sg, ss):
    c = lax.axis_index("c")
    s = lax.axis_index("s")
    base = s * ROWS_PER_TILE
    pltpu.sync_copy(zeros_hbm, acc.at[pl.ds(base, ROWS_PER_TILE)])
    plsc.subcore_barrier()

    def fire_g(j, k):
      pltpu.async_copy(tab_hbm.at[src_v.at[j]], rows[k], sg[k])

    def wait_g(j, k):
      pltpu.make_async_copy(tab_hbm.at[src_v.at[j]], rows[k], sg[k]).wait()

    def fire_s(j, k):
      return pltpu.async_copy(rows[k], acc.at[dst_v.at[j]], ss[k], add=True)

    def group_body(g, carry):
      pltpu.sync_copy(src_hbm.at[c, s, pl.ds(g * GROUP, GROUP)], src_v)
      pltpu.sync_copy(dst_hbm.at[c, s, pl.ds(g * GROUP, GROUP)], dst_v)
      for k in range(4):
        fire_g(k, k)

      def mbody(m, inner):
        j = 4 * m
        ds = []
        for k in range(4):
          wait_g(j + k, k)
          ds.append(fire_s(j + k, k))
        for k in range(4):
          ds[k].wait()
          fire_g(j + 4 + k, k)
        return inner

      lax.fori_loop(0, MSTEPS, mbody, carry)
      j = GROUP - 4
      ds = []
      for k in range(4):
        wait_g(j + k, k)
        ds.append(fire_s(j + k, k))
      for k in range(4):
        ds[k].wait()
      return carry

    lax.fori_loop(0, NGROUPS, group_body, 0)
    plsc.subcore_barrier()
    pltpu.sync_copy(acc.at[pl.ds(base, ROWS_PER_TILE)],
                    out_hbm.at[c, pl.ds(base, ROWS_PER_TILE)])

  return sc_agg


_sc_agg = _sc_agg_build()


def _tc_gru(x_ref, h_ref, aggx_ref, aggh_ref, wxroot_ref, wxrel_ref,
            whroot_ref, whrel_ref, b1_ref, b2_ref, o_ref):
  xb = x_ref[...]
  hb = h_ref[...]
  ax = aggx_ref[...]
  ah = aggh_ref[...]
  f32 = jnp.float32
  wx = (jnp.dot(xb, wxroot_ref[...], preferred_element_type=f32)
        + jnp.dot(ax, wxrel_ref[...], preferred_element_type=f32)
        + b1_ref[...])
  wh = (jnp.dot(hb, whroot_ref[...], preferred_element_type=f32)
        + jnp.dot(ah, whrel_ref[...], preferred_element_type=f32)
        + b2_ref[...])
  r = jax.nn.sigmoid(wx[:, :H] + wh[:, :H])
  z = jax.nn.sigmoid(wx[:, H:2 * H] + wh[:, H:2 * H])
  q = jnp.tanh(wx[:, 2 * H:] + r * wh[:, 2 * H:])
  o_ref[...] = (1.0 - z) * q + z * hb


def kernel(x, edge_index, h, Wx_rel, Wx_root, bx_rel, Wh_rel, Wh_root,
           bh_rel, bias):
  e = edge_index.shape[1]
  src = edge_index[0].astype(jnp.int32)
  dst = edge_index[1].astype(jnp.int32)
  pad = E_PAD - e
  src_p = jnp.concatenate([src, jnp.zeros((pad,), jnp.int32)])
  dst_p = jnp.concatenate([dst, jnp.full((pad,), N, jnp.int32)])
  dump = HALF + (jnp.arange(E_PAD, dtype=jnp.int32) % DUMP_SPREAD)
  src_cs, dst_cs = [], []
  for cc in range(NUM_CORES):
    in_range = (dst_p >= cc * HALF) & (dst_p < (cc + 1) * HALF)
    sp = jnp.where(in_range, 2 * src_p, 0)
    dl = jnp.where(in_range, dst_p - cc * HALF, dump)
    src_cs.append(jnp.stack([sp, sp + 1], axis=1).reshape(-1))
    dst_cs.append(jnp.stack([dl, ACC_HALF + dl], axis=1).reshape(-1))
  src4 = jnp.stack(src_cs).reshape(NUM_CORES, NUM_TILES, CHUNKS, CHUNK)
  dst4 = jnp.stack(dst_cs).reshape(NUM_CORES, NUM_TILES, CHUNKS, CHUNK)
  tab = jnp.concatenate([x, h], axis=1).reshape(2 * N, D)
  zeros = jnp.zeros((ROWS_PER_TILE, D), jnp.float32)

  agg = _sc_agg(tab, src4, dst4, zeros)
  aggx = jnp.concatenate([agg[0, :HALF], agg[1, :HALF]])
  aggh = jnp.concatenate(
      [agg[0, ACC_HALF:ACC_HALF + HALF], agg[1, ACC_HALF:ACC_HALF + HALF]])

  b1 = (bx_rel + bias).reshape(1, GATE)
  b2 = bh_rel.reshape(1, GATE)

  grid = (N // ROW_BLOCK,)
  h_new = pl.pallas_call(
      _tc_gru,
      grid=grid,
      in_specs=[
          pl.BlockSpec((ROW_BLOCK, D), lambda i: (i, 0)),
          pl.BlockSpec((ROW_BLOCK, D), lambda i: (i, 0)),
          pl.BlockSpec((ROW_BLOCK, D), lambda i: (i, 0)),
          pl.BlockSpec((ROW_BLOCK, D), lambda i: (i, 0)),
          pl.BlockSpec((D, GATE), lambda i: (0, 0)),
          pl.BlockSpec((D, GATE), lambda i: (0, 0)),
          pl.BlockSpec((D, GATE), lambda i: (0, 0)),
          pl.BlockSpec((D, GATE), lambda i: (0, 0)),
          pl.BlockSpec((1, GATE), lambda i: (0, 0)),
          pl.BlockSpec((1, GATE), lambda i: (0, 0)),
      ],
      out_specs=pl.BlockSpec((ROW_BLOCK, D), lambda i: (i, 0)),
      out_shape=jax.ShapeDtypeStruct((N, H), jnp.float32),
  )(x, h, aggx, aggh, Wx_root, Wx_rel, Wh_root, Wh_rel, b1, b2)
  return h_new


# speedup vs baseline: 7.3830x; 7.3830x over previous
import functools

import jax
import jax.numpy as jnp
from jax import lax
from jax.experimental import pallas as pl
from jax.experimental.pallas import tpu as pltpu
from jax.experimental.pallas import tpu_sc as plsc

N = 10000
D = 128
H = 128
GATE = 3 * H

NUM_CORES = 2
NUM_TILES = 16
HALF = N // 2
CHUNK = 64
CHUNKS = 640
GROUP = 40
NGROUPS = CHUNKS // GROUP
MSTEPS = (GROUP - 4) // 4
E_PAD = NUM_TILES * CHUNKS * CHUNK // 2
ACC_HALF = 5440
ACC_ROWS = 2 * ACC_HALF
ROWS_PER_TILE = ACC_ROWS // NUM_TILES
DUMP_SPREAD = 440
HOT_SPREAD = 32

ROW_BLOCK = 200
BLOCKS_PER_CORE = HALF // ROW_BLOCK
H_BLOCK_OFF = ACC_HALF // ROW_BLOCK


def _sc_agg_build():
  mesh = plsc.VectorSubcoreMesh(core_axis_name="c", subcore_axis_name="s")

  @functools.partial(
      pl.kernel,
      out_type=jax.ShapeDtypeStruct((NUM_CORES, ACC_ROWS, D), jnp.float32),
      mesh=mesh,
      scratch_types=[
          pltpu.VMEM_SHARED((ACC_ROWS, D), jnp.float32),
          pltpu.VMEM((GROUP, CHUNK), jnp.int32),
          pltpu.VMEM((GROUP, CHUNK), jnp.int32),
          [pltpu.VMEM((CHUNK, D), jnp.float32)] * 4,
          [pltpu.SemaphoreType.DMA] * 4,
          [pltpu.SemaphoreType.DMA] * 4,
      ],
  )
  def sc_agg(tab_hbm, src_hbm, dst_hbm, zeros_hbm, out_hbm,
             acc, src_v, dst_v, rows, sg, ss):
    c = lax.axis_index("c")
    s = lax.axis_index("s")
    base = s * ROWS_PER_TILE
    pltpu.sync_copy(zeros_hbm, acc.at[pl.ds(base, ROWS_PER_TILE)])
    plsc.subcore_barrier()

    def fire_g(j, k):
      pltpu.async_copy(tab_hbm.at[src_v.at[j]], rows[k], sg[k])

    def wait_g(j, k):
      pltpu.make_async_copy(tab_hbm.at[src_v.at[j]], rows[k], sg[k]).wait()

    def fire_s(j, k):
      return pltpu.async_copy(rows[k], acc.at[dst_v.at[j]], ss[k], add=True)

    def group_body(g, carry):
      pltpu.sync_copy(src_hbm.at[c, s, pl.ds(g * GROUP, GROUP)], src_v)
      pltpu.sync_copy(dst_hbm.at[c, s, pl.ds(g * GROUP, GROUP)], dst_v)
      for k in range(4):
        fire_g(k, k)

      def mbody(m, inner):
        j = 4 * m
        ds = []
        for k in range(4):
          wait_g(j + k, k)
          ds.append(fire_s(j + k, k))
        for k in range(4):
          ds[k].wait()
          fire_g(j + 4 + k, k)
        return inner

      lax.fori_loop(0, MSTEPS, mbody, carry)
      j = GROUP - 4
      ds = []
      for k in range(4):
        wait_g(j + k, k)
        ds.append(fire_s(j + k, k))
      for k in range(4):
        ds[k].wait()
      return carry

    lax.fori_loop(0, NGROUPS, group_body, 0)
    plsc.subcore_barrier()
    pltpu.sync_copy(acc.at[pl.ds(base, ROWS_PER_TILE)],
                    out_hbm.at[c, pl.ds(base, ROWS_PER_TILE)])

  return sc_agg


_sc_agg = _sc_agg_build()


def _tc_gru(x_ref, h_ref, aggx_ref, aggh_ref, wxroot_ref, wxrel_ref,
            whroot_ref, whrel_ref, b1_ref, b2_ref, o_ref):
  xb = x_ref[...]
  hb = h_ref[...]
  ax = aggx_ref[...]
  ah = aggh_ref[...]
  f32 = jnp.float32
  wx = (jnp.dot(xb, wxroot_ref[...], preferred_element_type=f32)
        + jnp.dot(ax, wxrel_ref[...], preferred_element_type=f32)
        + b1_ref[...])
  wh = (jnp.dot(hb, whroot_ref[...], preferred_element_type=f32)
        + jnp.dot(ah, whrel_ref[...], preferred_element_type=f32)
        + b2_ref[...])
  r = jax.nn.sigmoid(wx[:, :H] + wh[:, :H])
  z = jax.nn.sigmoid(wx[:, H:2 * H] + wh[:, H:2 * H])
  q = jnp.tanh(wx[:, 2 * H:] + r * wh[:, 2 * H:])
  o_ref[...] = (1.0 - z) * q + z * hb


def kernel(x, edge_index, h, Wx_rel, Wx_root, bx_rel, Wh_rel, Wh_root,
           bh_rel, bias):
  e = edge_index.shape[1]
  src = edge_index[0].astype(jnp.int32)
  dst = edge_index[1].astype(jnp.int32)
  pad = E_PAD - e
  src_p = jnp.concatenate([src, jnp.zeros((pad,), jnp.int32)])
  dst_p = jnp.concatenate([dst, jnp.full((pad,), N, jnp.int32)])
  epos = jnp.arange(E_PAD, dtype=jnp.int32)
  dump = HALF + (epos % DUMP_SPREAD)
  hot = epos % HOT_SPREAD
  src_cs, dst_cs = [], []
  for cc in range(NUM_CORES):
    in_range = (dst_p >= cc * HALF) & (dst_p < (cc + 1) * HALF)
    sp = jnp.where(in_range, 2 * src_p, 2 * hot)
    dl = jnp.where(in_range, dst_p - cc * HALF, dump)
    src_cs.append(jnp.stack([sp, sp + 1], axis=1).reshape(-1))
    dst_cs.append(jnp.stack([dl, ACC_HALF + dl], axis=1).reshape(-1))
  src4 = jnp.stack(src_cs).reshape(NUM_CORES, NUM_TILES, CHUNKS, CHUNK)
  dst4 = jnp.stack(dst_cs).reshape(NUM_CORES, NUM_TILES, CHUNKS, CHUNK)
  tab = jnp.concatenate([x, h], axis=1).reshape(2 * N, D)
  zeros = jnp.zeros((ROWS_PER_TILE, D), jnp.float32)

  agg = _sc_agg(tab, src4, dst4, zeros)
  aggx = jnp.concatenate([agg[0, :HALF], agg[1, :HALF]])
  aggh = jnp.concatenate(
      [agg[0, ACC_HALF:ACC_HALF + HALF], agg[1, ACC_HALF:ACC_HALF + HALF]])

  b1 = (bx_rel + bias).reshape(1, GATE)
  b2 = bh_rel.reshape(1, GATE)

  grid = (N // ROW_BLOCK,)
  h_new = pl.pallas_call(
      _tc_gru,
      grid=grid,
      in_specs=[
          pl.BlockSpec((ROW_BLOCK, D), lambda i: (i, 0)),
          pl.BlockSpec((ROW_BLOCK, D), lambda i: (i, 0)),
          pl.BlockSpec((ROW_BLOCK, D), lambda i: (i, 0)),
          pl.BlockSpec((ROW_BLOCK, D), lambda i: (i, 0)),
          pl.BlockSpec((D, GATE), lambda i: (0, 0)),
          pl.BlockSpec((D, GATE), lambda i: (0, 0)),
          pl.BlockSpec((D, GATE), lambda i: (0, 0)),
          pl.BlockSpec((D, GATE), lambda i: (0, 0)),
          pl.BlockSpec((1, GATE), lambda i: (0, 0)),
          pl.BlockSpec((1, GATE), lambda i: (0, 0)),
      ],
      out_specs=pl.BlockSpec((ROW_BLOCK, D), lambda i: (i, 0)),
      out_shape=jax.ShapeDtypeStruct((N, H), jnp.float32),
  )(x, h, aggx, aggh, Wx_root, Wx_rel, Wh_root, Wh_rel, b1, b2)
  return h_new


# speedup vs baseline: 20.2353x; 2.7408x over previous
import functools

import jax
import jax.numpy as jnp
from jax import lax
from jax.experimental import pallas as pl
from jax.experimental.pallas import tpu as pltpu
from jax.experimental.pallas import tpu_sc as plsc

N = 10000
D = 128
H = 128
GATE = 3 * H

NUM_CORES = 2
NUM_TILES = 16
CHUNK = 64
CHUNKS = 320
GROUP = 40
NGROUPS = CHUNKS // GROUP
MSTEPS = (GROUP - 4) // 4
E_PAD = NUM_TILES * CHUNKS * CHUNK
ROWS_PER_TILE = 640
N_PAD = NUM_TILES * ROWS_PER_TILE

ROW_BLOCK = 400


def _sc_agg_build():
  mesh = plsc.VectorSubcoreMesh(core_axis_name="c", subcore_axis_name="s")

  @functools.partial(
      pl.kernel,
      out_type=jax.ShapeDtypeStruct((NUM_CORES, N_PAD, D), jnp.float32),
      mesh=mesh,
      scratch_types=[
          pltpu.VMEM_SHARED((N_PAD, D), jnp.float32),
          pltpu.VMEM((GROUP, CHUNK), jnp.int32),
          pltpu.VMEM((GROUP, CHUNK), jnp.int32),
          [pltpu.VMEM((CHUNK, D), jnp.float32)] * 4,
          [pltpu.SemaphoreType.DMA] * 4,
          [pltpu.SemaphoreType.DMA] * 4,
      ],
  )
  def sc_agg(tab_hbm, src_hbm, dst_hbm, zeros_hbm, out_hbm,
             acc, src_v, dst_v, rows, sg, ss):
    c = lax.axis_index("c")
    s = lax.axis_index("s")
    base = s * ROWS_PER_TILE
    pltpu.sync_copy(zeros_hbm, acc.at[pl.ds(base, ROWS_PER_TILE)])
    plsc.subcore_barrier()

    def fire_g(j, k):
      pltpu.async_copy(tab_hbm.at[src_v.at[j]], rows[k], sg[k])

    def wait_g(j, k):
      pltpu.make_async_copy(tab_hbm.at[src_v.at[j]], rows[k], sg[k]).wait()

    def fire_s(j, k):
      return pltpu.async_copy(rows[k], acc.at[dst_v.at[j]], ss[k], add=True)

    def group_body(g, carry):
      pltpu.sync_copy(src_hbm.at[c, s, pl.ds(g * GROUP, GROUP)], src_v)
      pltpu.sync_copy(dst_hbm.at[s, pl.ds(g * GROUP, GROUP)], dst_v)
      for k in range(4):
        fire_g(k, k)

      def mbody(m, inner):
        j = 4 * m
        ds = []
        for k in range(4):
          wait_g(j + k, k)
          ds.append(fire_s(j + k, k))
        for k in range(4):
          ds[k].wait()
          fire_g(j + 4 + k, k)
        return inner

      lax.fori_loop(0, MSTEPS, mbody, carry)
      j = GROUP - 4
      ds = []
      for k in range(4):
        wait_g(j + k, k)
        ds.append(fire_s(j + k, k))
      for k in range(4):
        ds[k].wait()
      return carry

    lax.fori_loop(0, NGROUPS, group_body, 0)
    plsc.subcore_barrier()
    pltpu.sync_copy(acc.at[pl.ds(base, ROWS_PER_TILE)],
                    out_hbm.at[c, pl.ds(base, ROWS_PER_TILE)])

  return sc_agg


_sc_agg = _sc_agg_build()


def _tc_gru(x_ref, h_ref, agg_ref, wxroot_ref, wxrel_ref, whroot_ref,
            whrel_ref, b1_ref, b2_ref, o_ref):
  xb = x_ref[...]
  hb = h_ref[...]
  ax = agg_ref[0]
  ah = agg_ref[1]
  f32 = jnp.float32
  wx = (jnp.dot(xb, wxroot_ref[...], preferred_element_type=f32)
        + jnp.dot(ax, wxrel_ref[...], preferred_element_type=f32)
        + b1_ref[...])
  wh = (jnp.dot(hb, whroot_ref[...], preferred_element_type=f32)
        + jnp.dot(ah, whrel_ref[...], preferred_element_type=f32)
        + b2_ref[...])
  r = jax.nn.sigmoid(wx[:, :H] + wh[:, :H])
  z = jax.nn.sigmoid(wx[:, H:2 * H] + wh[:, H:2 * H])
  q = jnp.tanh(wx[:, 2 * H:] + r * wh[:, 2 * H:])
  o_ref[...] = (1.0 - z) * q + z * hb


def kernel(x, edge_index, h, Wx_rel, Wx_root, bx_rel, Wh_rel, Wh_root,
           bh_rel, bias):
  e = edge_index.shape[1]
  src = edge_index[0].astype(jnp.int32)
  dst = edge_index[1].astype(jnp.int32)
  pad = E_PAD - e
  src_p = jnp.concatenate([src, jnp.zeros((pad,), jnp.int32)])
  dst_p = jnp.concatenate([dst, jnp.full((pad,), N, jnp.int32)])
  src4 = jnp.stack([src_p, src_p + N]).reshape(NUM_CORES, NUM_TILES, CHUNKS,
                                               CHUNK)
  dst3 = dst_p.reshape(NUM_TILES, CHUNKS, CHUNK)
  tab = jnp.concatenate([x, h], axis=0)
  zeros = jnp.zeros((ROWS_PER_TILE, D), jnp.float32)

  agg = _sc_agg(tab, src4, dst3, zeros)

  b1 = (bx_rel + bias).reshape(1, GATE)
  b2 = bh_rel.reshape(1, GATE)

  grid = (N // ROW_BLOCK,)
  h_new = pl.pallas_call(
      _tc_gru,
      grid=grid,
      in_specs=[
          pl.BlockSpec((ROW_BLOCK, D), lambda i: (i, 0)),
          pl.BlockSpec((ROW_BLOCK, D), lambda i: (i, 0)),
          pl.BlockSpec((NUM_CORES, ROW_BLOCK, D), lambda i: (0, i, 0)),
          pl.BlockSpec((D, GATE), lambda i: (0, 0)),
          pl.BlockSpec((D, GATE), lambda i: (0, 0)),
          pl.BlockSpec((D, GATE), lambda i: (0, 0)),
          pl.BlockSpec((D, GATE), lambda i: (0, 0)),
          pl.BlockSpec((1, GATE), lambda i: (0, 0)),
          pl.BlockSpec((1, GATE), lambda i: (0, 0)),
      ],
      out_specs=pl.BlockSpec((ROW_BLOCK, D), lambda i: (i, 0)),
      out_shape=jax.ShapeDtypeStruct((N, H), jnp.float32),
  )(x, h, agg, Wx_root, Wx_rel, Wh_root, Wh_rel, b1, b2)
  return h_new
